# baseline (speedup 1.0000x reference)
import jax
import jax.numpy as jnp
from jax.experimental import pallas as pl
from jax.experimental.pallas import tpu as pltpu

_BT = 512


def _router_tile(x_ref, wt_ref, b_ref, probs_ref, logits_ref, zloss_ref):
    i = pl.program_id(0)
    j = pl.program_id(1)
    logits = jax.lax.dot_general(
        wt_ref[...], x_ref[0],
        (((1,), (1,)), ((), ())),
        preferred_element_type=jnp.float32,
    )
    logits = logits + b_ref[...]
    logits_ref[0] = logits
    m = jnp.max(logits, axis=0, keepdims=True)
    e = jnp.exp(logits - m)
    s = jnp.sum(e, axis=0, keepdims=True)
    probs_ref[0] = e * (1.0 / s)
    log_z = jnp.log(s) + m
    part = jnp.sum(log_z * log_z)

    @pl.when((i == 0) & (j == 0))
    def _init():
        zloss_ref[0] = 0.0

    zloss_ref[0] += part


@jax.jit
def _run(x, Wt, b):
    num_groups, tokens_per_group, hidden = x.shape
    num_experts = Wt.shape[0]
    grid = (num_groups, tokens_per_group // _BT)
    probs_t, logits_t, zsum = pl.pallas_call(
        _router_tile,
        grid=grid,
        in_specs=[
            pl.BlockSpec((1, _BT, hidden), lambda i, j: (i, j, 0)),
            pl.BlockSpec((num_experts, hidden), lambda i, j: (0, 0)),
            pl.BlockSpec((num_experts, 1), lambda i, j: (0, 0)),
        ],
        out_specs=[
            pl.BlockSpec((1, num_experts, _BT), lambda i, j: (i, 0, j)),
            pl.BlockSpec((1, num_experts, _BT), lambda i, j: (i, 0, j)),
            pl.BlockSpec(memory_space=pltpu.SMEM),
        ],
        out_shape=[
            jax.ShapeDtypeStruct((num_groups, num_experts, tokens_per_group), jnp.float32),
            jax.ShapeDtypeStruct((num_groups, num_experts, tokens_per_group), jnp.float32),
            jax.ShapeDtypeStruct((1,), jnp.float32),
        ],
    )(x, Wt, b)
    return probs_t, logits_t, zsum


def kernel(token_inputs, expert_capacity, W, b):
    del expert_capacity
    num_groups, tokens_per_group, _ = token_inputs.shape
    x = token_inputs.astype(jnp.float32)
    probs_t, logits_t, zsum = _run(x, W.astype(jnp.float32).T,
                                   b.astype(jnp.float32)[:, None])
    probs = jnp.transpose(probs_t, (0, 2, 1))
    logits = jnp.transpose(logits_t, (0, 2, 1))
    z_loss = zsum[0] / (num_groups * tokens_per_group)
    return probs, logits, z_loss


# speedup vs baseline: 1.0579x; 1.0579x over previous
import jax
import jax.numpy as jnp
from jax.experimental import pallas as pl
from jax.experimental.pallas import tpu as pltpu

_BT = 1024


def _router_tile(x_ref, wt_ref, b_ref, probs_ref, logits_ref, zloss_ref):
    i = pl.program_id(0)
    j = pl.program_id(1)
    logits = jax.lax.dot_general(
        wt_ref[...], x_ref[0],
        (((1,), (1,)), ((), ())),
        preferred_element_type=jnp.float32,
    )
    logits = logits + b_ref[...]
    logits_ref[0] = logits
    m = jnp.max(logits, axis=0, keepdims=True)
    e = jnp.exp(logits - m)
    s = jnp.sum(e, axis=0, keepdims=True)
    probs_ref[0] = e * (1.0 / s)
    log_z = jnp.log(s) + m
    n_tokens = pl.num_programs(0) * pl.num_programs(1) * log_z.shape[1]
    part = jnp.sum(log_z * log_z) * (1.0 / n_tokens)

    @pl.when((i == 0) & (j == 0))
    def _init():
        zloss_ref[0] = 0.0

    zloss_ref[0] += part


@jax.jit
def _run(x, Wt, b):
    num_groups, tokens_per_group, hidden = x.shape
    num_experts = Wt.shape[0]
    grid = (num_groups, tokens_per_group // _BT)
    probs_t, logits_t, zsum = pl.pallas_call(
        _router_tile,
        grid=grid,
        in_specs=[
            pl.BlockSpec((1, _BT, hidden), lambda i, j: (i, j, 0)),
            pl.BlockSpec((num_experts, hidden), lambda i, j: (0, 0)),
            pl.BlockSpec((num_experts, 1), lambda i, j: (0, 0)),
        ],
        out_specs=[
            pl.BlockSpec((1, num_experts, _BT), lambda i, j: (i, 0, j)),
            pl.BlockSpec((1, num_experts, _BT), lambda i, j: (i, 0, j)),
            pl.BlockSpec(memory_space=pltpu.SMEM),
        ],
        out_shape=[
            jax.ShapeDtypeStruct((num_groups, num_experts, tokens_per_group), jnp.float32),
            jax.ShapeDtypeStruct((num_groups, num_experts, tokens_per_group), jnp.float32),
            jax.ShapeDtypeStruct((1,), jnp.float32),
        ],
    )(x, Wt, b)
    return probs_t, logits_t, zsum


def kernel(token_inputs, expert_capacity, W, b):
    del expert_capacity
    num_groups, tokens_per_group, _ = token_inputs.shape
    x = token_inputs.astype(jnp.float32)
    probs_t, logits_t, zsum = _run(x, W.astype(jnp.float32).T,
                                   b.astype(jnp.float32)[:, None])
    probs = jnp.transpose(probs_t, (0, 2, 1))
    logits = jnp.transpose(logits_t, (0, 2, 1))
    return probs, logits, zsum[0]


# speedup vs baseline: 1.0581x; 1.0002x over previous
import jax
import jax.numpy as jnp
from jax.experimental import pallas as pl
from jax.experimental.pallas import tpu as pltpu

_BT = 1024
_BH = _BT // 2


def _router_tile(xa_ref, xb_ref, wt_ref, b_ref, probs_ref, logits_ref,
                 zloss_ref):
    i = pl.program_id(0)
    j = pl.program_id(1)
    dims = (((1,), (1,)), ((), ()))
    la = jax.lax.dot_general(wt_ref[...], xa_ref[0], dims,
                             preferred_element_type=jnp.float32)
    lb = jax.lax.dot_general(wt_ref[...], xb_ref[0], dims,
                             preferred_element_type=jnp.float32)
    logits = jnp.concatenate([la, lb], axis=1) + b_ref[...]
    logits_ref[0] = logits
    m = jnp.max(logits, axis=0, keepdims=True)
    e = jnp.exp(logits - m)
    s = jnp.sum(e, axis=0, keepdims=True)
    probs_ref[0] = e * (1.0 / s)
    log_z = jnp.log(s) + m
    n_tokens = pl.num_programs(0) * pl.num_programs(1) * log_z.shape[1]
    part = jnp.sum(log_z * log_z) * (1.0 / n_tokens)

    @pl.when((i == 0) & (j == 0))
    def _init():
        zloss_ref[0] = 0.0

    zloss_ref[0] += part


@jax.jit
def _run(x, Wt, b):
    num_groups, tokens_per_group, hidden = x.shape
    num_experts = Wt.shape[0]
    grid = (num_groups, tokens_per_group // _BT)
    probs_t, logits_t, zsum = pl.pallas_call(
        _router_tile,
        grid=grid,
        in_specs=[
            pl.BlockSpec((1, _BH, hidden), lambda i, j: (i, 2 * j, 0)),
            pl.BlockSpec((1, _BH, hidden), lambda i, j: (i, 2 * j + 1, 0)),
            pl.BlockSpec((num_experts, hidden), lambda i, j: (0, 0)),
            pl.BlockSpec((num_experts, 1), lambda i, j: (0, 0)),
        ],
        out_specs=[
            pl.BlockSpec((1, num_experts, _BT), lambda i, j: (i, 0, j)),
            pl.BlockSpec((1, num_experts, _BT), lambda i, j: (i, 0, j)),
            pl.BlockSpec(memory_space=pltpu.SMEM),
        ],
        out_shape=[
            jax.ShapeDtypeStruct((num_groups, num_experts, tokens_per_group), jnp.float32),
            jax.ShapeDtypeStruct((num_groups, num_experts, tokens_per_group), jnp.float32),
            jax.ShapeDtypeStruct((1,), jnp.float32),
        ],
    )(x, x, Wt, b)
    return probs_t, logits_t, zsum


def kernel(token_inputs, expert_capacity, W, b):
    del expert_capacity
    num_groups, tokens_per_group, _ = token_inputs.shape
    x = token_inputs.astype(jnp.float32)
    probs_t, logits_t, zsum = _run(x, W.astype(jnp.float32).T,
                                   b.astype(jnp.float32)[:, None])
    probs = jnp.transpose(probs_t, (0, 2, 1))
    logits = jnp.transpose(logits_t, (0, 2, 1))
    return probs, logits, zsum[0]
